# baseline (speedup 1.0000x reference)
import functools

import jax
import jax.numpy as jnp
from jax.experimental import pallas as pl
from jax.experimental.pallas import tpu as pltpu

N = 8192
TILE_I = 512
TILE_J = 2048


def _cdist_body(a_ref, b_ref, o_ref):
    a = a_ref[...]
    b = b_ref[...]
    dot = jax.lax.dot_general(a, b, (((1,), (1,)), ((), ())),
                              preferred_element_type=jnp.float32)
    sqa = jnp.sum(a * a, axis=1, keepdims=True)
    sqb = jnp.sum(b * b, axis=1, keepdims=True)
    d2 = sqa + sqb.T - 2.0 * dot
    d2 = jnp.maximum(d2, 0.0)
    safe = jnp.where(d2 > 0, d2, 1.0)
    o_ref[...] = jnp.where(d2 > 0, jnp.sqrt(safe), 0.0)


def _cdist(z):
    n = z.shape[0]
    zp = jnp.pad(z, ((0, 0), (0, 128 - z.shape[1])))
    return pl.pallas_call(
        _cdist_body,
        grid=(n // TILE_I, n // TILE_J),
        in_specs=[
            pl.BlockSpec((TILE_I, 128), lambda i, j: (i, 0)),
            pl.BlockSpec((TILE_J, 128), lambda i, j: (j, 0)),
        ],
        out_specs=pl.BlockSpec((TILE_I, TILE_J), lambda i, j: (i, j)),
        out_shape=jax.ShapeDtypeStruct((n, n), jnp.float32),
    )(zp, zp)


def _gat_conv(x, src, dst, n, W, a_src, a_dst, bias, heads, out_ch, concat):
    xl = (x @ W).reshape(n, heads, out_ch)
    al_s = (xl * a_src[None]).sum(-1)
    al_d = (xl * a_dst[None]).sum(-1)
    e = jax.nn.leaky_relu(al_s[src] + al_d[dst], 0.2)
    m = jax.ops.segment_max(e, dst, num_segments=n)
    e = jnp.exp(e - m[dst])
    s = jax.ops.segment_sum(e, dst, num_segments=n)
    alpha = e / (s[dst] + 1e-16)
    out = jax.ops.segment_sum(xl[src] * alpha[:, :, None], dst, num_segments=n)
    if concat:
        out = out.reshape(n, heads * out_ch)
    else:
        out = out.mean(axis=1)
    return out + bias


def kernel(x, edge_index, W1, as1, ad1, b1, W2, as2, ad2, b2, W3, as3, ad3, b3):
    n = x.shape[0]
    loop = jnp.arange(n, dtype=edge_index.dtype)
    src = jnp.concatenate([edge_index[0], loop])
    dst = jnp.concatenate([edge_index[1], loop])
    h = _gat_conv(x, src, dst, n, W1, as1, ad1, b1, 2, 128, True)
    h = jax.nn.leaky_relu(h, 0.01)
    h = _gat_conv(h, src, dst, n, W2, as2, ad2, b2, 2, 64, True)
    h = jax.nn.leaky_relu(h, 0.01)
    h = _gat_conv(h, src, dst, n, W3, as3, ad3, b3, 1, 3, False)
    return _cdist(h)
